# Initial kernel scaffold; baseline (speedup 1.0000x reference)
import jax
import jax.numpy as jnp
from jax.experimental import pallas as pl


def kernel(verts, bds, faces, pix_to_face):
    raise NotImplementedError("write your pallas kernel here")


# speedup vs baseline: 364.3827x; 364.3827x over previous
import jax
import jax.numpy as jnp
from jax import lax
from jax.experimental import pallas as pl
from jax.experimental.pallas import tpu as pltpu
from jax.experimental.pallas import tpu_sc as plsc

BT, NV, NB, NF = 16, 6890, 5000, 13776
NSAMP, NSAMP_PAD = 1000, 1024
NPIX = BT * 256 * 256
NFACET = BT * NF
NC, NS = 2, 16
NW = NC * NS
FPT = 6912
NFACE_PAD = NW * FPT
FROWS = FPT // 128
PROWS = NPIX // 128
PROWS_T = PROWS // NS
PC = 256
NVTOT = BT * NV
NV_PAD = NS * FPT
TRASH = NVTOT
NVT = 2304
NV_TC = 3 * NVT
BIG = 3.0e38


def _vis_body(pix_hbm, fg0_hbm, fg1_hbm, fg2_hbm, out_hbm,
              pix_v, fg_v, fv_v, idx_v, ones_v, zeros_v, fv_sh, vis_sh):
    c = lax.axis_index("c")
    s = lax.axis_index("s")
    w = c * NS + s

    zf = jnp.zeros((16,), jnp.float32)
    of = jnp.ones((16,), jnp.float32)

    def _fill_zeros(i, carry):
        zeros_v[pl.ds(i * 16, 16)] = zf
        return carry

    lax.fori_loop(0, FPT // 16, _fill_zeros, 0)
    for q in range(8):
        ones_v[pl.ds(q * 16, 16)] = of

    for h in range(2):
        pltpu.sync_copy(zeros_v, fv_sh.at[pl.ds((s * 2 + h) * FPT, FPT)])
    pltpu.sync_copy(zeros_v, vis_sh.at[pl.ds(s * FPT, FPT)])
    plsc.subcore_barrier()

    for h in range(PROWS_T // PC):
        pltpu.sync_copy(pix_hbm.at[pl.ds(s * PROWS_T + h * PC, PC), :], pix_v)

        def _scat_face(j, carry):
            pltpu.sync_copy(ones_v, fv_sh.at[pix_v.at[j]], add=True)
            return carry

        lax.fori_loop(0, PC, _scat_face, 0)
    plsc.subcore_barrier()

    lo = w * FPT
    pltpu.sync_copy(fv_sh.at[pl.ds(lo, FPT)], fv_v)
    tv = jnp.full((16,), TRASH, jnp.int32)
    for fgk_hbm in (fg0_hbm, fg1_hbm, fg2_hbm):
        pltpu.sync_copy(fgk_hbm.at[pl.ds(lo, FPT)], fg_v)

        def _build(r, carry):
            for q in range(8):
                fv16 = fv_v[pl.ds(r * 128 + q * 16, 16)]
                g16 = fg_v[pl.ds(r * 128 + q * 16, 16)]
                idx_v[r, pl.ds(q * 16, 16)] = jnp.where(fv16 > 0.0, g16, tv)
            return carry

        lax.fori_loop(0, FROWS, _build, 0)

        def _scat_vert(j, carry):
            pltpu.sync_copy(ones_v, vis_sh.at[idx_v.at[j]], add=True)
            return carry

        lax.fori_loop(0, FROWS, _scat_vert, 0)
    plsc.subcore_barrier()

    pltpu.sync_copy(vis_sh.at[pl.ds(s * FPT, FPT)],
                    out_hbm.at[pl.ds(c * NV_PAD + s * FPT, FPT)])


_vis_call = pl.kernel(
    _vis_body,
    out_type=jax.ShapeDtypeStruct((NC * NV_PAD,), jnp.float32),
    mesh=plsc.VectorSubcoreMesh(core_axis_name="c", subcore_axis_name="s"),
    scratch_types=[
        pltpu.VMEM((PC, 128), jnp.int32),
        pltpu.VMEM((FPT,), jnp.int32),
        pltpu.VMEM((FPT,), jnp.float32),
        pltpu.VMEM((FROWS, 128), jnp.int32),
        pltpu.VMEM((128,), jnp.float32),
        pltpu.VMEM((FPT,), jnp.float32),
        pltpu.VMEM_SHARED((NFACE_PAD,), jnp.float32),
        pltpu.VMEM_SHARED((NV_PAD,), jnp.float32),
    ],
)


def _tc_body(bv_ref, bm_ref, vt_ref, va_ref, vb_ref, out_ref, acc_ref):
    j = pl.program_id(1)
    bv = bv_ref[0]
    vt = vt_ref[0]
    ab = jnp.dot(bv, vt, preferred_element_type=jnp.float32)
    a2 = jnp.sum(bv * bv, axis=1, keepdims=True)
    b2 = jnp.sum(vt * vt, axis=0, keepdims=True)
    dist = jnp.maximum(a2 + b2 - 2.0 * ab, 0.0)
    vis = va_ref[0] + vb_ref[0]
    masked = jnp.where(vis > 0.0, dist, 1000.0)
    col = j * NVT + lax.broadcasted_iota(jnp.int32, (1, NVT), 1)
    masked = jnp.where(col < NV, masked, BIG)
    rowmin = jnp.min(masked, axis=1, keepdims=True)

    @pl.when(j == 0)
    def _():
        acc_ref[...] = rowmin

    @pl.when(j > 0)
    def _():
        acc_ref[...] = jnp.minimum(acc_ref[...], rowmin)

    @pl.when(j == NV_TC // NVT - 1)
    def _():
        out_ref[0, 0, 0] = jnp.sum(acc_ref[...][:, 0] * bm_ref[0, 0])


_tc_call = pl.pallas_call(
    _tc_body,
    grid=(BT, NV_TC // NVT),
    in_specs=[
        pl.BlockSpec((1, NSAMP_PAD, 8), lambda b, j: (b, 0, 0)),
        pl.BlockSpec((1, 1, NSAMP_PAD), lambda b, j: (b, 0, 0)),
        pl.BlockSpec((1, 8, NVT), lambda b, j: (b, 0, j)),
        pl.BlockSpec((1, 1, NVT), lambda b, j: (b, 0, j)),
        pl.BlockSpec((1, 1, NVT), lambda b, j: (b, 0, j)),
    ],
    out_specs=pl.BlockSpec((1, 1, 1), lambda b, j: (b, 0, 0),
                           memory_space=pltpu.SMEM),
    out_shape=jax.ShapeDtypeStruct((BT, 1, 1), jnp.float32),
    scratch_shapes=[pltpu.VMEM((NSAMP_PAD, 1), jnp.float32)],
)


def kernel(verts, bds, faces, pix_to_face):
    perm = jax.random.permutation(jax.random.key(42), NB)[:NSAMP]
    bsel = bds[:, perm, :]
    bv = jnp.pad(bsel[..., :3], ((0, 0), (0, NSAMP_PAD - NSAMP), (0, 5)))
    bm = jnp.pad(bsel[..., 3], ((0, 0), (0, NSAMP_PAD - NSAMP)))
    bm = bm.reshape(BT, 1, NSAMP_PAD)
    vt = jnp.pad(verts.transpose(0, 2, 1), ((0, 0), (0, 5), (0, NV_TC - NV)))
    fg = faces + (jnp.arange(BT, dtype=jnp.int32) * NV)[:, None, None]
    fg = fg.reshape(-1, 3).T
    fg = jnp.pad(fg, ((0, 0), (0, NFACE_PAD - NFACET)))
    pix2d = pix_to_face.reshape(PROWS, 128)
    vis2 = _vis_call(pix2d, fg[0], fg[1], fg[2]).reshape(NC, NV_PAD)
    va = jnp.pad(vis2[0, :NVTOT].reshape(BT, 1, NV),
                 ((0, 0), (0, 0), (0, NV_TC - NV)))
    vb = jnp.pad(vis2[1, :NVTOT].reshape(BT, 1, NV),
                 ((0, 0), (0, 0), (0, NV_TC - NV)))
    loss16 = _tc_call(bv, bm, vt, va, vb)
    return jnp.mean(loss16)


# speedup vs baseline: 415.6470x; 1.1407x over previous
import jax
import jax.numpy as jnp
from jax import lax
from jax.experimental import pallas as pl
from jax.experimental.pallas import tpu as pltpu
from jax.experimental.pallas import tpu_sc as plsc

BT, NV, NB, NF = 16, 6890, 5000, 13776
NSAMP, NSAMP_PAD = 1000, 1024
NPIX = BT * 256 * 256
NFACET = BT * NF
NC, NS = 2, 16
NW = NC * NS
FPT = 6912
NFACE_PAD = NW * FPT
FPT2 = NFACE_PAD // NS
F2ROWS = FPT2 // 128
PROWS = NPIX // 128
PIXROWS_T = PROWS // NW
NVTOT = BT * NV
NV_PAD = NS * FPT
TRASH = NVTOT
NVT = 2304
NV_TC = 3 * NVT
BIGSQ = 1.0e36


def _vis_body(pix_hbm, fg0_hbm, fg1_hbm, fg2_hbm, out_hbm,
              pix_v, fg_v, fv_v, idx_v, ones_v, zeros_v, sem, fv_sh, vis_sh):
    c = lax.axis_index("c")
    s = lax.axis_index("s")
    w = c * NS + s

    zf = jnp.zeros((16,), jnp.float32)
    of = jnp.ones((16,), jnp.float32)

    def _fill_zeros(i, carry):
        zeros_v[pl.ds(i * 16, 16)] = zf
        return carry

    lax.fori_loop(0, FPT // 16, _fill_zeros, 0)
    for q in range(8):
        ones_v[pl.ds(q * 16, 16)] = of

    for h in range(2):
        pltpu.sync_copy(zeros_v, fv_sh.at[pl.ds((s * 2 + h) * FPT, FPT)])
    pltpu.sync_copy(zeros_v, vis_sh.at[pl.ds(s * FPT, FPT)])
    plsc.subcore_barrier()

    pltpu.sync_copy(pix_hbm.at[pl.ds(w * PIXROWS_T, PIXROWS_T), :], pix_v)

    def _scat_face(o, carry):
        descs = [pltpu.async_copy(ones_v, fv_sh.at[pix_v.at[o * 8 + q]],
                                  sem, add=True) for q in range(8)]
        for d in descs:
            d.wait()
        return carry

    lax.fori_loop(0, PIXROWS_T // 8, _scat_face, 0)
    plsc.subcore_barrier()

    lo = s * FPT2
    pltpu.sync_copy(fv_sh.at[pl.ds(lo, FPT2)], fv_v)
    tv = jnp.full((16,), TRASH, jnp.int32)
    for fgk_hbm in (fg0_hbm, fg1_hbm, fg2_hbm):
        pltpu.sync_copy(fgk_hbm.at[pl.ds(lo, FPT2)], fg_v)

        def _build(r, carry):
            for q in range(8):
                fv16 = fv_v[pl.ds(r * 128 + q * 16, 16)]
                g16 = fg_v[pl.ds(r * 128 + q * 16, 16)]
                idx_v[r, pl.ds(q * 16, 16)] = jnp.where(fv16 > 0.0, g16, tv)
            return carry

        lax.fori_loop(0, F2ROWS, _build, 0)

        def _scat_vert(o, carry):
            descs = [pltpu.async_copy(ones_v, vis_sh.at[idx_v.at[o * 9 + q]],
                                      sem, add=True) for q in range(9)]
            for d in descs:
                d.wait()
            return carry

        lax.fori_loop(0, F2ROWS // 9, _scat_vert, 0)
    plsc.subcore_barrier()

    pltpu.sync_copy(vis_sh.at[pl.ds(s * FPT, FPT)],
                    out_hbm.at[pl.ds(c * NV_PAD + s * FPT, FPT)])


_vis_call = pl.kernel(
    _vis_body,
    out_type=jax.ShapeDtypeStruct((NC * NV_PAD,), jnp.float32),
    mesh=plsc.VectorSubcoreMesh(core_axis_name="c", subcore_axis_name="s"),
    scratch_types=[
        pltpu.VMEM((PIXROWS_T, 128), jnp.int32),
        pltpu.VMEM((FPT2,), jnp.int32),
        pltpu.VMEM((FPT2,), jnp.float32),
        pltpu.VMEM((F2ROWS, 128), jnp.int32),
        pltpu.VMEM((128,), jnp.float32),
        pltpu.VMEM((FPT,), jnp.float32),
        pltpu.SemaphoreType.DMA,
        pltpu.VMEM_SHARED((NFACE_PAD,), jnp.float32),
        pltpu.VMEM_SHARED((NV_PAD,), jnp.float32),
    ],
)


def _tc_body(bv_ref, bm_ref, q_ref, vt_ref, bpen_ref, out_ref, acc_ref):
    j = pl.program_id(1)
    bv = bv_ref[0]
    ab2 = jnp.dot(bv, vt_ref[0], preferred_element_type=jnp.float32)
    e = ab2 + bpen_ref[0]
    rowmin = jnp.min(e, axis=1, keepdims=True)

    @pl.when(j == 0)
    def _():
        acc_ref[...] = rowmin

    @pl.when(j > 0)
    def _():
        acc_ref[...] = jnp.minimum(acc_ref[...], rowmin)

    @pl.when(j == NV_TC // NVT - 1)
    def _():
        a2 = jnp.sum(bv * bv, axis=1)
        q = q_ref[0, 0, 0]
        mind = jnp.maximum(jnp.minimum(acc_ref[...][:, 0] + a2, q), 0.0)
        out_ref[0, 0, 0] = jnp.sum(mind * bm_ref[0, 0])


_tc_call = pl.pallas_call(
    _tc_body,
    grid=(BT, NV_TC // NVT),
    in_specs=[
        pl.BlockSpec((1, NSAMP_PAD, 4), lambda b, j: (b, 0, 0)),
        pl.BlockSpec((1, 1, NSAMP_PAD), lambda b, j: (b, 0, 0)),
        pl.BlockSpec((1, 1, 1), lambda b, j: (b, 0, 0),
                     memory_space=pltpu.SMEM),
        pl.BlockSpec((1, 4, NVT), lambda b, j: (b, 0, j)),
        pl.BlockSpec((1, 1, NVT), lambda b, j: (b, 0, j)),
    ],
    out_specs=pl.BlockSpec((1, 1, 1), lambda b, j: (b, 0, 0),
                           memory_space=pltpu.SMEM),
    out_shape=jax.ShapeDtypeStruct((BT, 1, 1), jnp.float32),
    scratch_shapes=[pltpu.VMEM((NSAMP_PAD, 1), jnp.float32)],
)


def kernel(verts, bds, faces, pix_to_face):
    perm = jax.random.permutation(jax.random.key(42), NB)[:NSAMP]
    bsel = bds[:, perm, :]
    bv = jnp.pad(bsel[..., :3],
                 ((0, 0), (0, NSAMP_PAD - NSAMP), (0, 1)))
    bm = jnp.pad(bsel[..., 3], ((0, 0), (0, NSAMP_PAD - NSAMP)))
    bm = bm.reshape(BT, 1, NSAMP_PAD)
    fg = faces + (jnp.arange(BT, dtype=jnp.int32) * NV)[:, None, None]
    fg = fg.reshape(-1, 3).T
    fg = jnp.pad(fg, ((0, 0), (0, NFACE_PAD - NFACET)))
    pix2d = pix_to_face.reshape(PROWS, 128)
    vis2 = _vis_call(pix2d, fg[0], fg[1], fg[2]).reshape(NC, NV_PAD)
    visb = (vis2[0, :NVTOT] + vis2[1, :NVTOT]).reshape(BT, NV) > 0.0
    q = jnp.where(jnp.all(visb, axis=1), 3.0e38, 1000.0)
    q = q.reshape(BT, 1, 1)
    vsq = jnp.sum(verts * verts, axis=-1)
    bpen = jnp.where(visb, vsq, 1.0e30)
    bpen = jnp.pad(bpen.reshape(BT, 1, NV),
                   ((0, 0), (0, 0), (0, NV_TC - NV)), constant_values=BIGSQ)
    vt = jnp.pad(-2.0 * verts.transpose(0, 2, 1),
                 ((0, 0), (0, 1), (0, NV_TC - NV)))
    loss16 = _tc_call(bv, bm, q, vt, bpen)
    return jnp.mean(loss16)


# speedup vs baseline: 492.1661x; 1.1841x over previous
import jax
import jax.numpy as jnp
from jax import lax
from jax.experimental import pallas as pl
from jax.experimental.pallas import tpu as pltpu
from jax.experimental.pallas import tpu_sc as plsc

BT, NV, NB, NF = 16, 6890, 5000, 13776
NSAMP, NSAMP_PAD = 1000, 1024
NPIX = BT * 256 * 256
NFACET = BT * NF
NC, NS = 2, 16
NW = NC * NS
FPT = 6912
NFACE_PAD = NW * FPT
FROWS = FPT // 128
PROWS = NPIX // 128
PIXROWS_T = PROWS // NW
PC = 256
NVTOT = BT * NV
NV_PAD = NS * FPT
TRASH = NVTOT
NVT = 2304
NV_TC = 3 * NVT
BIGSQ = 1.0e36


def _vis_body(pix_hbm, fg0_hbm, fg1_hbm, fg2_hbm, out_hbm,
              pix_v, fg_v, fv_v, idx_v, ones_v, zeros_v, sem, fv_sh, vis_sh):
    c = lax.axis_index("c")
    s = lax.axis_index("s")
    w = c * NS + s

    zf = jnp.zeros((16,), jnp.float32)
    of = jnp.ones((16,), jnp.float32)

    def _fill_zeros(i, carry):
        zeros_v[pl.ds(i * 16, 16)] = zf
        return carry

    lax.fori_loop(0, FPT // 16, _fill_zeros, 0)
    for q in range(8):
        ones_v[pl.ds(q * 16, 16)] = of

    for h in range(2):
        pltpu.sync_copy(zeros_v, fv_sh.at[pl.ds((s * 2 + h) * FPT, FPT)])
    pltpu.sync_copy(zeros_v, vis_sh.at[pl.ds(s * FPT, FPT)])
    plsc.subcore_barrier()

    for h in range(2):
        pltpu.sync_copy(pix_hbm.at[pl.ds((s * 2 + h) * PC, PC), :], pix_v)

        def _scat_face(o, carry):
            descs = [pltpu.async_copy(ones_v, fv_sh.at[pix_v.at[o * 8 + q]],
                                      sem, add=True) for q in range(8)]
            for d in descs:
                d.wait()
            return carry

        lax.fori_loop(0, PC // 8, _scat_face, 0)
    plsc.subcore_barrier()

    lo = w * FPT
    pltpu.sync_copy(fv_sh.at[pl.ds(lo, FPT)], fv_v)
    tv = jnp.full((16,), TRASH, jnp.int32)
    for fgk_hbm in (fg0_hbm, fg1_hbm, fg2_hbm):
        pltpu.sync_copy(fgk_hbm.at[pl.ds(lo, FPT)], fg_v)

        def _build(r, carry):
            for q in range(8):
                fv16 = fv_v[pl.ds(r * 128 + q * 16, 16)]
                g16 = fg_v[pl.ds(r * 128 + q * 16, 16)]
                idx_v[r, pl.ds(q * 16, 16)] = jnp.where(fv16 > 0.0, g16, tv)
            return carry

        lax.fori_loop(0, FROWS, _build, 0)

        def _scat_vert(o, carry):
            descs = [pltpu.async_copy(ones_v, vis_sh.at[idx_v.at[o * 9 + q]],
                                      sem, add=True) for q in range(9)]
            for d in descs:
                d.wait()
            return carry

        lax.fori_loop(0, FROWS // 9, _scat_vert, 0)
    plsc.subcore_barrier()

    pltpu.sync_copy(vis_sh.at[pl.ds(s * FPT, FPT)],
                    out_hbm.at[pl.ds(c * NV_PAD + s * FPT, FPT)])


_vis_call = pl.kernel(
    _vis_body,
    out_type=jax.ShapeDtypeStruct((NC * NV_PAD,), jnp.float32),
    mesh=plsc.VectorSubcoreMesh(core_axis_name="c", subcore_axis_name="s"),
    scratch_types=[
        pltpu.VMEM((PC, 128), jnp.int32),
        pltpu.VMEM((FPT,), jnp.int32),
        pltpu.VMEM((FPT,), jnp.float32),
        pltpu.VMEM((FROWS, 128), jnp.int32),
        pltpu.VMEM((128,), jnp.float32),
        pltpu.VMEM((FPT,), jnp.float32),
        pltpu.SemaphoreType.DMA,
        pltpu.VMEM_SHARED((NFACE_PAD,), jnp.float32),
        pltpu.VMEM_SHARED((NV_PAD,), jnp.float32),
    ],
)


def _tc_body(bv_ref, bm_ref, q_ref, vt_ref, bpen_ref, out_ref, acc_ref):
    j = pl.program_id(1)
    bv = bv_ref[0]
    ab2 = jnp.dot(bv, vt_ref[0], preferred_element_type=jnp.float32)
    e = ab2 + bpen_ref[0]
    rowmin = jnp.min(e, axis=1, keepdims=True)

    @pl.when(j == 0)
    def _():
        acc_ref[...] = rowmin

    @pl.when(j > 0)
    def _():
        acc_ref[...] = jnp.minimum(acc_ref[...], rowmin)

    @pl.when(j == NV_TC // NVT - 1)
    def _():
        a2 = jnp.sum(bv * bv, axis=1)
        q = q_ref[0, 0, 0]
        mind = jnp.maximum(jnp.minimum(acc_ref[...][:, 0] + a2, q), 0.0)
        out_ref[0, 0, 0] = jnp.sum(mind * bm_ref[0, 0])


_tc_call = pl.pallas_call(
    _tc_body,
    grid=(BT, NV_TC // NVT),
    in_specs=[
        pl.BlockSpec((1, NSAMP_PAD, 4), lambda b, j: (b, 0, 0)),
        pl.BlockSpec((1, 1, NSAMP_PAD), lambda b, j: (b, 0, 0)),
        pl.BlockSpec((1, 1, 1), lambda b, j: (b, 0, 0),
                     memory_space=pltpu.SMEM),
        pl.BlockSpec((1, 4, NVT), lambda b, j: (b, 0, j)),
        pl.BlockSpec((1, 1, NVT), lambda b, j: (b, 0, j)),
    ],
    out_specs=pl.BlockSpec((1, 1, 1), lambda b, j: (b, 0, 0),
                           memory_space=pltpu.SMEM),
    out_shape=jax.ShapeDtypeStruct((BT, 1, 1), jnp.float32),
    scratch_shapes=[pltpu.VMEM((NSAMP_PAD, 1), jnp.float32)],
)


def kernel(verts, bds, faces, pix_to_face):
    perm = jax.random.permutation(jax.random.key(42), NB)[:NSAMP]
    bsel = bds[:, perm, :]
    bv = jnp.pad(bsel[..., :3],
                 ((0, 0), (0, NSAMP_PAD - NSAMP), (0, 1)))
    bm = jnp.pad(bsel[..., 3], ((0, 0), (0, NSAMP_PAD - NSAMP)))
    bm = bm.reshape(BT, 1, NSAMP_PAD)
    fg = faces + (jnp.arange(BT, dtype=jnp.int32) * NV)[:, None, None]
    fg = fg.reshape(-1, 3).T
    fg = jnp.pad(fg, ((0, 0), (0, NFACE_PAD - NFACET)))
    pix2d = pix_to_face.reshape(PROWS, 128)
    vis2 = _vis_call(pix2d, fg[0], fg[1], fg[2]).reshape(NC, NV_PAD)
    visb = (vis2[0, :NVTOT] + vis2[1, :NVTOT]).reshape(BT, NV) > 0.0
    q = jnp.where(jnp.all(visb, axis=1), 3.0e38, 1000.0)
    q = q.reshape(BT, 1, 1)
    vsq = jnp.sum(verts * verts, axis=-1)
    bpen = jnp.where(visb, vsq, 1.0e30)
    bpen = jnp.pad(bpen.reshape(BT, 1, NV),
                   ((0, 0), (0, 0), (0, NV_TC - NV)), constant_values=BIGSQ)
    vt = jnp.pad(-2.0 * verts.transpose(0, 2, 1),
                 ((0, 0), (0, 1), (0, NV_TC - NV)))
    loss16 = _tc_call(bv, bm, q, vt, bpen)
    return jnp.mean(loss16)


# speedup vs baseline: 496.1219x; 1.0080x over previous
import jax
import jax.numpy as jnp
from jax import lax
from jax.experimental import pallas as pl
from jax.experimental.pallas import tpu as pltpu
from jax.experimental.pallas import tpu_sc as plsc

BT, NV, NB, NF = 16, 6890, 5000, 13776
NSAMP, NSAMP_PAD = 1000, 1024
NPIX = BT * 256 * 256
NFACET = BT * NF
NC, NS = 2, 16
NW = NC * NS
FPT = 6912
NFACE_PAD = NW * FPT
FROWS = FPT // 128
PROWS = NPIX // 128
PIXROWS_T = PROWS // NW
PC = 256
NVTOT = BT * NV
NV_PAD = NS * FPT
TRASH = NVTOT
NVT = 6912
NV_TC = 1 * NVT
BIGSQ = 1.0e36


def _vis_body(pix_hbm, fg0_hbm, fg1_hbm, fg2_hbm, out_hbm,
              pix_v, fg_v, fv_v, idx_v, ones_v, zeros_v, sem, fv_sh, vis_sh):
    c = lax.axis_index("c")
    s = lax.axis_index("s")
    w = c * NS + s

    zf = jnp.zeros((16,), jnp.float32)
    of = jnp.ones((16,), jnp.float32)

    def _fill_zeros(i, carry):
        zeros_v[pl.ds(i * 16, 16)] = zf
        return carry

    lax.fori_loop(0, FPT // 16, _fill_zeros, 0)
    for q in range(8):
        ones_v[pl.ds(q * 16, 16)] = of

    for h in range(2):
        pltpu.sync_copy(zeros_v, fv_sh.at[pl.ds((s * 2 + h) * FPT, FPT)])
    pltpu.sync_copy(zeros_v, vis_sh.at[pl.ds(s * FPT, FPT)])
    plsc.subcore_barrier()

    for h in range(2):
        pltpu.sync_copy(pix_hbm.at[pl.ds((s * 2 + h) * PC, PC), :], pix_v)

        def _scat_face(o, carry):
            descs = [pltpu.async_copy(ones_v, fv_sh.at[pix_v.at[o * 8 + q]],
                                      sem, add=True) for q in range(8)]
            for d in descs:
                d.wait()
            return carry

        lax.fori_loop(0, PC // 8, _scat_face, 0)
    plsc.subcore_barrier()

    lo = w * FPT
    pltpu.sync_copy(fv_sh.at[pl.ds(lo, FPT)], fv_v)
    tv = jnp.full((16,), TRASH, jnp.int32)
    for fgk_hbm in (fg0_hbm, fg1_hbm, fg2_hbm):
        pltpu.sync_copy(fgk_hbm.at[pl.ds(lo, FPT)], fg_v)

        def _build(r, carry):
            for q in range(8):
                fv16 = fv_v[pl.ds(r * 128 + q * 16, 16)]
                g16 = fg_v[pl.ds(r * 128 + q * 16, 16)]
                idx_v[r, pl.ds(q * 16, 16)] = jnp.where(fv16 > 0.0, g16, tv)
            return carry

        lax.fori_loop(0, FROWS, _build, 0)

        def _scat_vert(o, carry):
            descs = [pltpu.async_copy(ones_v, vis_sh.at[idx_v.at[o * 9 + q]],
                                      sem, add=True) for q in range(9)]
            for d in descs:
                d.wait()
            return carry

        lax.fori_loop(0, FROWS // 9, _scat_vert, 0)
    plsc.subcore_barrier()

    pltpu.sync_copy(vis_sh.at[pl.ds(s * FPT, FPT)],
                    out_hbm.at[pl.ds(c * NV_PAD + s * FPT, FPT)])


_vis_call = pl.kernel(
    _vis_body,
    out_type=jax.ShapeDtypeStruct((NC * NV_PAD,), jnp.float32),
    mesh=plsc.VectorSubcoreMesh(core_axis_name="c", subcore_axis_name="s"),
    scratch_types=[
        pltpu.VMEM((PC, 128), jnp.int32),
        pltpu.VMEM((FPT,), jnp.int32),
        pltpu.VMEM((FPT,), jnp.float32),
        pltpu.VMEM((FROWS, 128), jnp.int32),
        pltpu.VMEM((128,), jnp.float32),
        pltpu.VMEM((FPT,), jnp.float32),
        pltpu.SemaphoreType.DMA,
        pltpu.VMEM_SHARED((NFACE_PAD,), jnp.float32),
        pltpu.VMEM_SHARED((NV_PAD,), jnp.float32),
    ],
)


def _tc_body(bv_ref, bm_ref, q_ref, vt_ref, bpen_ref, out_ref, acc_ref):
    j = pl.program_id(1)
    bv = bv_ref[0]
    ab2 = jnp.dot(bv, vt_ref[0], preferred_element_type=jnp.float32)
    e = ab2 + bpen_ref[0]
    rowmin = jnp.min(e, axis=1, keepdims=True)

    @pl.when(j == 0)
    def _():
        acc_ref[...] = rowmin

    @pl.when(j > 0)
    def _():
        acc_ref[...] = jnp.minimum(acc_ref[...], rowmin)

    @pl.when(j == NV_TC // NVT - 1)
    def _():
        a2 = jnp.sum(bv * bv, axis=1)
        q = q_ref[0, 0, 0]
        mind = jnp.maximum(jnp.minimum(acc_ref[...][:, 0] + a2, q), 0.0)
        out_ref[0, 0, 0] = jnp.sum(mind * bm_ref[0, 0])


_tc_call = pl.pallas_call(
    _tc_body,
    grid=(BT, NV_TC // NVT),
    in_specs=[
        pl.BlockSpec((1, NSAMP_PAD, 4), lambda b, j: (b, 0, 0)),
        pl.BlockSpec((1, 1, NSAMP_PAD), lambda b, j: (b, 0, 0)),
        pl.BlockSpec((1, 1, 1), lambda b, j: (b, 0, 0),
                     memory_space=pltpu.SMEM),
        pl.BlockSpec((1, 4, NVT), lambda b, j: (b, 0, j)),
        pl.BlockSpec((1, 1, NVT), lambda b, j: (b, 0, j)),
    ],
    out_specs=pl.BlockSpec((1, 1, 1), lambda b, j: (b, 0, 0),
                           memory_space=pltpu.SMEM),
    out_shape=jax.ShapeDtypeStruct((BT, 1, 1), jnp.float32),
    scratch_shapes=[pltpu.VMEM((NSAMP_PAD, 1), jnp.float32)],
)


def kernel(verts, bds, faces, pix_to_face):
    perm = jax.random.permutation(jax.random.key(42), NB)[:NSAMP]
    bsel = bds[:, perm, :]
    bv = jnp.pad(bsel[..., :3],
                 ((0, 0), (0, NSAMP_PAD - NSAMP), (0, 1)))
    bm = jnp.pad(bsel[..., 3], ((0, 0), (0, NSAMP_PAD - NSAMP)))
    bm = bm.reshape(BT, 1, NSAMP_PAD)
    fg = faces + (jnp.arange(BT, dtype=jnp.int32) * NV)[:, None, None]
    fg = fg.reshape(-1, 3).T
    fg = jnp.pad(fg, ((0, 0), (0, NFACE_PAD - NFACET)))
    pix2d = pix_to_face.reshape(PROWS, 128)
    vis2 = _vis_call(pix2d, fg[0], fg[1], fg[2]).reshape(NC, NV_PAD)
    visb = (vis2[0, :NVTOT] + vis2[1, :NVTOT]).reshape(BT, NV) > 0.0
    q = jnp.where(jnp.all(visb, axis=1), 3.0e38, 1000.0)
    q = q.reshape(BT, 1, 1)
    vsq = jnp.sum(verts * verts, axis=-1)
    bpen = jnp.where(visb, vsq, 1.0e30)
    bpen = jnp.pad(bpen.reshape(BT, 1, NV),
                   ((0, 0), (0, 0), (0, NV_TC - NV)), constant_values=BIGSQ)
    vt = jnp.pad(-2.0 * verts.transpose(0, 2, 1),
                 ((0, 0), (0, 1), (0, NV_TC - NV)))
    loss16 = _tc_call(bv, bm, q, vt, bpen)
    return jnp.mean(loss16)


# speedup vs baseline: 533.9072x; 1.0762x over previous
import jax
import jax.numpy as jnp
import numpy as np
from jax import lax
from jax.experimental import pallas as pl
from jax.experimental.pallas import tpu as pltpu
from jax.experimental.pallas import tpu_sc as plsc

BT, NV, NB, NF = 16, 6890, 5000, 13776
NSAMP, NSAMP_PAD = 1000, 1024
NPIX = BT * 256 * 256
NFACET = BT * NF
NC, NS = 2, 16
NW = NC * NS
FPT = 6912
NFACE_PAD = NW * FPT
FROWS = FPT // 128
PROWS = NPIX // 128
PIXROWS_T = PROWS // NW
PC = 256
NVTOT = BT * NV
NV_PAD = NS * FPT
TRASH = NVTOT
NVT = 6912
NV_TC = 1 * NVT
BIGSQ = 1.0e36

with jax.default_device(jax.devices("cpu")[0]):
    _PERM = np.asarray(
        jax.random.permutation(jax.random.key(42), NB)[:NSAMP])


def _vis_body(pix_hbm, fg0_hbm, fg1_hbm, fg2_hbm, out_hbm,
              pix_v, fg_v, fv_v, idx_v, ones_v, zeros_v, sem, fv_sh, vis_sh):
    c = lax.axis_index("c")
    s = lax.axis_index("s")
    w = c * NS + s

    zf = jnp.zeros((16,), jnp.float32)
    of = jnp.ones((16,), jnp.float32)

    def _fill_zeros(i, carry):
        zeros_v[pl.ds(i * 16, 16)] = zf
        return carry

    lax.fori_loop(0, FPT // 16, _fill_zeros, 0)
    for q in range(8):
        ones_v[pl.ds(q * 16, 16)] = of

    for h in range(2):
        pltpu.sync_copy(zeros_v, fv_sh.at[pl.ds((s * 2 + h) * FPT, FPT)])
    pltpu.sync_copy(zeros_v, vis_sh.at[pl.ds(s * FPT, FPT)])
    plsc.subcore_barrier()

    for h in range(2):
        pltpu.sync_copy(pix_hbm.at[pl.ds((s * 2 + h) * PC, PC), :], pix_v)

        def _scat_face(o, carry):
            descs = [pltpu.async_copy(ones_v, fv_sh.at[pix_v.at[o * 8 + q]],
                                      sem, add=True) for q in range(8)]
            for d in descs:
                d.wait()
            return carry

        lax.fori_loop(0, PC // 8, _scat_face, 0)
    plsc.subcore_barrier()

    lo = w * FPT
    pltpu.sync_copy(fv_sh.at[pl.ds(lo, FPT)], fv_v)
    tv = jnp.full((16,), TRASH, jnp.int32)
    for fgk_hbm in (fg0_hbm, fg1_hbm, fg2_hbm):
        pltpu.sync_copy(fgk_hbm.at[pl.ds(lo, FPT)], fg_v)

        def _build(r, carry):
            for q in range(8):
                fv16 = fv_v[pl.ds(r * 128 + q * 16, 16)]
                g16 = fg_v[pl.ds(r * 128 + q * 16, 16)]
                idx_v[r, pl.ds(q * 16, 16)] = jnp.where(fv16 > 0.0, g16, tv)
            return carry

        lax.fori_loop(0, FROWS, _build, 0)

        def _scat_vert(o, carry):
            descs = [pltpu.async_copy(ones_v, vis_sh.at[idx_v.at[o * 9 + q]],
                                      sem, add=True) for q in range(9)]
            for d in descs:
                d.wait()
            return carry

        lax.fori_loop(0, FROWS // 9, _scat_vert, 0)
    plsc.subcore_barrier()

    pltpu.sync_copy(vis_sh.at[pl.ds(s * FPT, FPT)],
                    out_hbm.at[pl.ds(c * NV_PAD + s * FPT, FPT)])


_vis_call = pl.kernel(
    _vis_body,
    out_type=jax.ShapeDtypeStruct((NC * NV_PAD,), jnp.float32),
    mesh=plsc.VectorSubcoreMesh(core_axis_name="c", subcore_axis_name="s"),
    scratch_types=[
        pltpu.VMEM((PC, 128), jnp.int32),
        pltpu.VMEM((FPT,), jnp.int32),
        pltpu.VMEM((FPT,), jnp.float32),
        pltpu.VMEM((FROWS, 128), jnp.int32),
        pltpu.VMEM((128,), jnp.float32),
        pltpu.VMEM((FPT,), jnp.float32),
        pltpu.SemaphoreType.DMA,
        pltpu.VMEM_SHARED((NFACE_PAD,), jnp.float32),
        pltpu.VMEM_SHARED((NV_PAD,), jnp.float32),
    ],
)


def _tc_body(bv_ref, bm_ref, q_ref, vt_ref, bpen_ref, out_ref, acc_ref):
    j = pl.program_id(1)
    bv = bv_ref[0]
    ab2 = jnp.dot(bv, vt_ref[0], preferred_element_type=jnp.float32)
    e = ab2 + bpen_ref[0]
    rowmin = jnp.min(e, axis=1, keepdims=True)

    @pl.when(j == 0)
    def _():
        acc_ref[...] = rowmin

    @pl.when(j > 0)
    def _():
        acc_ref[...] = jnp.minimum(acc_ref[...], rowmin)

    @pl.when(j == NV_TC // NVT - 1)
    def _():
        a2 = jnp.sum(bv * bv, axis=1)
        q = q_ref[0, 0, 0]
        mind = jnp.maximum(jnp.minimum(acc_ref[...][:, 0] + a2, q), 0.0)
        out_ref[0, 0, 0] = jnp.sum(mind * bm_ref[0, 0])


_tc_call = pl.pallas_call(
    _tc_body,
    grid=(BT, NV_TC // NVT),
    in_specs=[
        pl.BlockSpec((1, NSAMP_PAD, 4), lambda b, j: (b, 0, 0)),
        pl.BlockSpec((1, 1, NSAMP_PAD), lambda b, j: (b, 0, 0)),
        pl.BlockSpec((1, 1, 1), lambda b, j: (b, 0, 0),
                     memory_space=pltpu.SMEM),
        pl.BlockSpec((1, 4, NVT), lambda b, j: (b, 0, j)),
        pl.BlockSpec((1, 1, NVT), lambda b, j: (b, 0, j)),
    ],
    out_specs=pl.BlockSpec((1, 1, 1), lambda b, j: (b, 0, 0),
                           memory_space=pltpu.SMEM),
    out_shape=jax.ShapeDtypeStruct((BT, 1, 1), jnp.float32),
    scratch_shapes=[pltpu.VMEM((NSAMP_PAD, 1), jnp.float32)],
)


def kernel(verts, bds, faces, pix_to_face):
    bsel = bds[:, _PERM, :]
    bv = jnp.pad(bsel[..., :3],
                 ((0, 0), (0, NSAMP_PAD - NSAMP), (0, 1)))
    bm = jnp.pad(bsel[..., 3], ((0, 0), (0, NSAMP_PAD - NSAMP)))
    bm = bm.reshape(BT, 1, NSAMP_PAD)
    fg = faces + (jnp.arange(BT, dtype=jnp.int32) * NV)[:, None, None]
    fg = fg.reshape(-1, 3).T
    fg = jnp.pad(fg, ((0, 0), (0, NFACE_PAD - NFACET)))
    pix2d = pix_to_face.reshape(PROWS, 128)
    vis2 = _vis_call(pix2d, fg[0], fg[1], fg[2]).reshape(NC, NV_PAD)
    visb = (vis2[0, :NVTOT] + vis2[1, :NVTOT]).reshape(BT, NV) > 0.0
    q = jnp.where(jnp.all(visb, axis=1), 3.0e38, 1000.0)
    q = q.reshape(BT, 1, 1)
    vsq = jnp.sum(verts * verts, axis=-1)
    bpen = jnp.where(visb, vsq, 1.0e30)
    bpen = jnp.pad(bpen.reshape(BT, 1, NV),
                   ((0, 0), (0, 0), (0, NV_TC - NV)), constant_values=BIGSQ)
    vt = jnp.pad(-2.0 * verts.transpose(0, 2, 1),
                 ((0, 0), (0, 1), (0, NV_TC - NV)))
    loss16 = _tc_call(bv, bm, q, vt, bpen)
    return jnp.mean(loss16)


# speedup vs baseline: 536.5554x; 1.0050x over previous
import jax
import jax.numpy as jnp
import numpy as np
from jax import lax
from jax.experimental import pallas as pl
from jax.experimental.pallas import tpu as pltpu
from jax.experimental.pallas import tpu_sc as plsc

BT, NV, NB, NF = 16, 6890, 5000, 13776
NSAMP, NSAMP_PAD = 1000, 1024
NPIX = BT * 256 * 256
NFACET = BT * NF
NC, NS = 2, 16
NW = NC * NS
FPT = 6912
NFACE_PAD = NW * FPT
FROWS = FPT // 128
PROWS = NPIX // 128
PIXROWS_T = PROWS // NW
PC = 256
NVTOT = BT * NV
NV_PAD = NS * FPT
TRASH = NV_PAD
NVT = 6912
NV_TC = 1 * NVT
BIGSQ = 1.0e36

with jax.default_device(jax.devices("cpu")[0]):
    _PERM = np.asarray(
        jax.random.permutation(jax.random.key(42), NB)[:NSAMP])


def _vis_body(pix_hbm, fg0_hbm, fg1_hbm, fg2_hbm, out_hbm,
              pix_v, fg_v, fv_v, idx_v, ones_v, zeros_v, sem, fv_sh, vis_sh):
    c = lax.axis_index("c")
    s = lax.axis_index("s")
    w = c * NS + s

    zf = jnp.zeros((16,), jnp.float32)
    of = jnp.ones((16,), jnp.float32)

    def _fill_zeros(i, carry):
        zeros_v[pl.ds(i * 16, 16)] = zf
        return carry

    lax.fori_loop(0, FPT // 16, _fill_zeros, 0)
    for q in range(8):
        ones_v[pl.ds(q * 16, 16)] = of

    for h in range(2):
        pltpu.sync_copy(zeros_v, fv_sh.at[pl.ds((s * 2 + h) * FPT, FPT)])
    pltpu.sync_copy(zeros_v, vis_sh.at[pl.ds(s * FPT, FPT)])
    plsc.subcore_barrier()

    for h in range(2):
        pltpu.sync_copy(pix_hbm.at[pl.ds((s * 2 + h) * PC, PC), :], pix_v)

        def _scat_face(o, carry):
            descs = [pltpu.async_copy(ones_v, fv_sh.at[pix_v.at[o * 8 + q]],
                                      sem, add=True) for q in range(8)]
            for d in descs:
                d.wait()
            return carry

        lax.fori_loop(0, PC // 8, _scat_face, 0)
    plsc.subcore_barrier()

    lo = w * FPT
    pltpu.sync_copy(fv_sh.at[pl.ds(lo, FPT)], fv_v)
    tv = jnp.full((16,), TRASH, jnp.int32)
    for fgk_hbm in (fg0_hbm, fg1_hbm, fg2_hbm):
        pltpu.sync_copy(fgk_hbm.at[pl.ds(lo, FPT)], fg_v)

        def _build(r, carry):
            for q in range(8):
                fv16 = fv_v[pl.ds(r * 128 + q * 16, 16)]
                g16 = fg_v[pl.ds(r * 128 + q * 16, 16)]
                idx_v[r, pl.ds(q * 16, 16)] = jnp.where(fv16 > 0.0, g16, tv)
            return carry

        lax.fori_loop(0, FROWS, _build, 0)

        def _scat_vert(o, carry):
            descs = [pltpu.async_copy(ones_v, vis_sh.at[idx_v.at[o * 9 + q]],
                                      sem, add=True) for q in range(9)]
            for d in descs:
                d.wait()
            return carry

        lax.fori_loop(0, FROWS // 9, _scat_vert, 0)
    plsc.subcore_barrier()

    pltpu.sync_copy(vis_sh.at[pl.ds(s * FPT, FPT)],
                    out_hbm.at[pl.ds(c * NV_PAD + s * FPT, FPT)])


_vis_call = pl.kernel(
    _vis_body,
    out_type=jax.ShapeDtypeStruct((NC * NV_PAD,), jnp.float32),
    mesh=plsc.VectorSubcoreMesh(core_axis_name="c", subcore_axis_name="s"),
    scratch_types=[
        pltpu.VMEM((PC, 128), jnp.int32),
        pltpu.VMEM((FPT,), jnp.int32),
        pltpu.VMEM((FPT,), jnp.float32),
        pltpu.VMEM((FROWS, 128), jnp.int32),
        pltpu.VMEM((128,), jnp.float32),
        pltpu.VMEM((FPT,), jnp.float32),
        pltpu.SemaphoreType.DMA,
        pltpu.VMEM_SHARED((NFACE_PAD,), jnp.float32),
        pltpu.VMEM_SHARED((NV_PAD + 128,), jnp.float32),
    ],
)


def _tc_body(bv_ref, bm_ref, q_ref, vt_ref, bpen_ref, out_ref, acc_ref):
    j = pl.program_id(1)
    bv = bv_ref[0]
    ab2 = jnp.dot(bv, vt_ref[0], preferred_element_type=jnp.float32)
    e = ab2 + bpen_ref[0]
    rowmin = jnp.min(e, axis=1, keepdims=True)

    @pl.when(j == 0)
    def _():
        acc_ref[...] = rowmin

    @pl.when(j > 0)
    def _():
        acc_ref[...] = jnp.minimum(acc_ref[...], rowmin)

    @pl.when(j == NV_TC // NVT - 1)
    def _():
        a2 = jnp.sum(bv * bv, axis=1)
        q = q_ref[0, 0, 0]
        mind = jnp.maximum(jnp.minimum(acc_ref[...][:, 0] + a2, q), 0.0)
        out_ref[0, 0, 0] = jnp.sum(mind * bm_ref[0, 0])


_tc_call = pl.pallas_call(
    _tc_body,
    grid=(BT, NV_TC // NVT),
    in_specs=[
        pl.BlockSpec((1, NSAMP_PAD, 4), lambda b, j: (b, 0, 0)),
        pl.BlockSpec((1, 1, NSAMP_PAD), lambda b, j: (b, 0, 0)),
        pl.BlockSpec((1, 1, 1), lambda b, j: (b, 0, 0),
                     memory_space=pltpu.SMEM),
        pl.BlockSpec((1, 4, NVT), lambda b, j: (b, 0, j)),
        pl.BlockSpec((1, 1, NVT), lambda b, j: (b, 0, j)),
    ],
    out_specs=pl.BlockSpec((1, 1, 1), lambda b, j: (b, 0, 0),
                           memory_space=pltpu.SMEM),
    out_shape=jax.ShapeDtypeStruct((BT, 1, 1), jnp.float32),
    scratch_shapes=[pltpu.VMEM((NSAMP_PAD, 1), jnp.float32)],
)


def kernel(verts, bds, faces, pix_to_face):
    bsel = bds[:, _PERM, :]
    bv = jnp.pad(bsel[..., :3],
                 ((0, 0), (0, NSAMP_PAD - NSAMP), (0, 1)))
    bm = jnp.pad(bsel[..., 3], ((0, 0), (0, NSAMP_PAD - NSAMP)))
    bm = bm.reshape(BT, 1, NSAMP_PAD)
    fg = faces + (jnp.arange(BT, dtype=jnp.int32) * NV_TC)[:, None, None]
    fg = fg.reshape(-1, 3).T
    fg = jnp.pad(fg, ((0, 0), (0, NFACE_PAD - NFACET)))
    pix2d = pix_to_face.reshape(PROWS, 128)
    vis2 = _vis_call(pix2d, fg[0], fg[1], fg[2]).reshape(NC, NV_PAD)
    visb = (vis2[0] + vis2[1]).reshape(BT, NV_TC) > 0.0
    q = jnp.where(jnp.all(visb[:, :NV], axis=1), 3.0e38, 1000.0)
    q = q.reshape(BT, 1, 1)
    vsq = jnp.pad(jnp.sum(verts * verts, axis=-1),
                  ((0, 0), (0, NV_TC - NV)))
    bpen = jnp.where(visb, vsq, 1.0e30)
    bpen = bpen.reshape(BT, 1, NV_TC)
    vt = jnp.pad(-2.0 * verts.transpose(0, 2, 1),
                 ((0, 0), (0, 1), (0, NV_TC - NV)))
    loss16 = _tc_call(bv, bm, q, vt, bpen)
    return jnp.mean(loss16)
